# baseline (speedup 1.0000x reference)
import functools

import jax
import jax.numpy as jnp
from jax import lax
from jax.experimental import pallas as pl
from jax.experimental.pallas import tpu as pltpu
from jax.experimental.pallas import tpu_sc as plsc

P1_FEATURE_CUTOFF = 24576
FT_DIM = 128
BLK = 4096

_DNT = (((1,), (1,)), ((), ()))


def _make_sc_gather(V, D, B):
    info = plsc.get_sparse_core_info()
    NC, NS = info.num_cores, info.num_subcores
    NW = NC * NS
    assert B % (8 * NW) == 0 and D % info.num_lanes == 0
    b_per_w = B // NW
    mesh = plsc.VectorSubcoreMesh(core_axis_name="c", subcore_axis_name="s")

    @functools.partial(
        pl.kernel,
        mesh=mesh,
        out_type=jax.ShapeDtypeStruct((B, D), jnp.float32),
        scratch_types=[
            pltpu.VMEM((b_per_w,), jnp.int32),
            pltpu.VMEM((b_per_w, D), jnp.float32),
            pltpu.SemaphoreType.DMA,
        ],
    )
    def sc_gather(table_hbm, idx_hbm, out_hbm, idx_v, rows_v, sem):
        wid = lax.axis_index("s") * NC + lax.axis_index("c")
        base = wid * b_per_w
        pltpu.sync_copy(idx_hbm.at[pl.ds(base, b_per_w)], idx_v)
        pltpu.async_copy(table_hbm.at[idx_v], rows_v, sem).wait()
        pltpu.sync_copy(rows_v, out_hbm.at[pl.ds(base, b_per_w)])

    return sc_gather


def _mlp_body(g_ref, f_ref, d_ref, w0_ref, bft_ref, w1_ref, b1_ref,
              w2_ref, b2_ref, w3_ref, b3_ref, out_ref):
    bf = jnp.bfloat16
    f32 = jnp.float32
    w0 = w0_ref[...]
    c = 2.0 * w0 + bft_ref[...]
    r0 = jnp.maximum(w0 + c, 0.0).astype(bf)
    hg = jnp.maximum(g_ref[...] + c, 0.0).astype(bf)
    is_p1 = f_ref[...] != 0
    h_stm = jnp.where(is_p1, hg, r0)
    h_nstm = jnp.where(is_p1, r0, hg)
    xcat = jnp.concatenate([h_stm, h_nstm, d_ref[...]], axis=1)
    x1 = lax.dot_general(xcat, w1_ref[...], _DNT, preferred_element_type=f32)
    h1 = jnp.maximum(x1 + b1_ref[...], 0.0)
    h2 = jnp.maximum(
        lax.dot_general(h1.astype(bf), w2_ref[...], _DNT,
                        preferred_element_type=f32) + b2_ref[...], 0.0)
    x3t = lax.dot_general(w3_ref[...], h2.astype(bf), _DNT,
                          preferred_element_type=f32)
    out_ref[...] = jnp.tanh(x3t + b3_ref[...])


def _mlp_call(g, f2d, dense, w0, bft, w1, b1, w2, b2, w3, b3):
    B = g.shape[0]
    H = w1.shape[0]
    TI = w1.shape[1]
    H2 = w2.shape[0]
    DD = dense.shape[1]
    grid = (B // BLK,)
    rep = lambda i: (0, 0)
    return pl.pallas_call(
        _mlp_body,
        grid=grid,
        in_specs=[
            pl.BlockSpec((BLK, FT_DIM), lambda i: (i, 0)),
            pl.BlockSpec((BLK, 1), lambda i: (i, 0)),
            pl.BlockSpec((BLK, DD), lambda i: (i, 0)),
            pl.BlockSpec((1, FT_DIM), rep),
            pl.BlockSpec((1, FT_DIM), rep),
            pl.BlockSpec((H, TI), rep),
            pl.BlockSpec((1, H), rep),
            pl.BlockSpec((H2, H), rep),
            pl.BlockSpec((1, H2), rep),
            pl.BlockSpec((1, H2), rep),
            pl.BlockSpec((1, 1), rep),
        ],
        out_specs=pl.BlockSpec((1, BLK), lambda i: (0, i)),
        out_shape=jax.ShapeDtypeStruct((1, B), jnp.float32),
    )(g, f2d, dense, w0, bft, w1, b1, w2, b2, w3, b3)


def kernel(sparse_batch, dense_batch, W_ft, b_ft, W1, b1, W2, b2, W3, b3):
    B = sparse_batch.shape[0]
    si = sparse_batch.astype(jnp.int32)
    f = si[:, 0]
    m8 = (f < P1_FEATURE_CUTOFF).astype(jnp.int8)[:, None]

    bf = jnp.bfloat16
    dense_bf = dense_batch.astype(bf)
    w1bf = W1.astype(bf)
    w2bf = W2.astype(bf)
    w3bf = W3.astype(bf)
    w0 = W_ft[0:1, :]
    bft = b_ft[None, :]
    b1r = b1[None, :]
    b2r = b2[None, :]
    b3r = b3.reshape(1, 1)

    NCH = 2
    CH = B // NCH
    sc_gather = _make_sc_gather(W_ft.shape[0], FT_DIM, CH)
    outs = []
    for i in range(NCH):
        sl = slice(i * CH, (i + 1) * CH)
        g = sc_gather(W_ft, f[sl])
        outs.append(_mlp_call(
            g, m8[sl], dense_bf[sl], w0, bft,
            w1bf, b1r, w2bf, b2r, w3bf, b3r,
        ))
    return jnp.concatenate(outs, axis=1)[0]


# speedup vs baseline: 1.2274x; 1.2274x over previous
import functools

import jax
import jax.numpy as jnp
from jax import lax
from jax.experimental import pallas as pl
from jax.experimental.pallas import tpu as pltpu
from jax.experimental.pallas import tpu_sc as plsc

P1_FEATURE_CUTOFF = 24576
FT_DIM = 128
BLK = 4096

_DNT = (((1,), (1,)), ((), ()))


def _make_sc_gather(V, D, B):
    info = plsc.get_sparse_core_info()
    NC, NS = info.num_cores, info.num_subcores
    NW = NC * NS
    assert B % (8 * NW) == 0 and D % info.num_lanes == 0
    b_per_w = B // NW
    mesh = plsc.VectorSubcoreMesh(core_axis_name="c", subcore_axis_name="s")

    @functools.partial(
        pl.kernel,
        mesh=mesh,
        out_type=jax.ShapeDtypeStruct((B, D), jnp.float32),
        scratch_types=[
            pltpu.VMEM((b_per_w,), jnp.int32),
            pltpu.VMEM((b_per_w, D), jnp.float32),
            pltpu.SemaphoreType.DMA,
        ],
    )
    def sc_gather(table_hbm, idx_hbm, out_hbm, idx_v, rows_v, sem):
        wid = lax.axis_index("s") * NC + lax.axis_index("c")
        base = wid * b_per_w
        pltpu.sync_copy(idx_hbm.at[pl.ds(base, b_per_w)], idx_v)
        pltpu.async_copy(table_hbm.at[idx_v], rows_v, sem).wait()
        pltpu.sync_copy(rows_v, out_hbm.at[pl.ds(base, b_per_w)])

    return sc_gather


def _mlp_body(g_ref, f_ref, d_ref, w0_ref, bft_ref, w1_ref, b1_ref,
              w2_ref, b2_ref, w3_ref, b3_ref, out_ref):
    bf = jnp.bfloat16
    f32 = jnp.float32
    w0 = w0_ref[...]
    c = 2.0 * w0 + bft_ref[...]
    r0 = jnp.maximum(w0 + c, 0.0).astype(bf)
    hg = jnp.maximum(g_ref[...] + c, 0.0).astype(bf)
    is_p1 = f_ref[...] != 0
    h_stm = jnp.where(is_p1, hg, r0)
    h_nstm = jnp.where(is_p1, r0, hg)
    xcat = jnp.concatenate([h_stm, h_nstm, d_ref[...]], axis=1)
    x1 = lax.dot_general(xcat, w1_ref[...], _DNT, preferred_element_type=f32)
    h1 = jnp.maximum(x1 + b1_ref[...], 0.0)
    h2 = jnp.maximum(
        lax.dot_general(h1.astype(bf), w2_ref[...], _DNT,
                        preferred_element_type=f32) + b2_ref[...], 0.0)
    x3t = lax.dot_general(w3_ref[...], h2.astype(bf), _DNT,
                          preferred_element_type=f32)
    out_ref[...] = jnp.tanh(x3t + b3_ref[...])


def _mlp_call(g, f2d, dense, w0, bft, w1, b1, w2, b2, w3, b3):
    B = g.shape[0]
    H = w1.shape[0]
    TI = w1.shape[1]
    H2 = w2.shape[0]
    DD = dense.shape[1]
    grid = (B // BLK,)
    rep = lambda i: (0, 0)
    return pl.pallas_call(
        _mlp_body,
        grid=grid,
        in_specs=[
            pl.BlockSpec((BLK, FT_DIM), lambda i: (i, 0)),
            pl.BlockSpec((BLK, 1), lambda i: (i, 0)),
            pl.BlockSpec((BLK, DD), lambda i: (i, 0)),
            pl.BlockSpec((1, FT_DIM), rep),
            pl.BlockSpec((1, FT_DIM), rep),
            pl.BlockSpec((H, TI), rep),
            pl.BlockSpec((1, H), rep),
            pl.BlockSpec((H2, H), rep),
            pl.BlockSpec((1, H2), rep),
            pl.BlockSpec((1, H2), rep),
            pl.BlockSpec((1, 1), rep),
        ],
        out_specs=pl.BlockSpec((1, BLK), lambda i: (0, i)),
        out_shape=jax.ShapeDtypeStruct((1, B), jnp.float32),
    )(g, f2d, dense, w0, bft, w1, b1, w2, b2, w3, b3)


def kernel(sparse_batch, dense_batch, W_ft, b_ft, W1, b1, W2, b2, W3, b3):
    B = sparse_batch.shape[0]
    si = sparse_batch.astype(jnp.int32)
    f = si[:, 0]
    m8 = (f < P1_FEATURE_CUTOFF).astype(jnp.int8)[:, None]

    bf = jnp.bfloat16
    dense_bf = dense_batch.astype(bf)
    w1bf = W1.astype(bf)
    w2bf = W2.astype(bf)
    w3bf = W3.astype(bf)
    w0 = W_ft[0:1, :]
    bft = b_ft[None, :]
    b1r = b1[None, :]
    b2r = b2[None, :]
    b3r = b3.reshape(1, 1)

    sc_gather = _make_sc_gather(W_ft.shape[0], FT_DIM, B)
    g = sc_gather(W_ft, f)
    out = _mlp_call(
        g, m8, dense_bf, w0, bft, w1bf, b1r, w2bf, b2r, w3bf, b3r,
    )
    return out[0]


# speedup vs baseline: 1.2326x; 1.0042x over previous
import functools

import jax
import jax.numpy as jnp
from jax import lax
from jax.experimental import pallas as pl
from jax.experimental.pallas import tpu as pltpu
from jax.experimental.pallas import tpu_sc as plsc

P1_FEATURE_CUTOFF = 24576
FT_DIM = 128
BLK = 4096

_DNT = (((1,), (1,)), ((), ()))


def _make_sc_gather(V, D, B):
    info = plsc.get_sparse_core_info()
    NC, NS = info.num_cores, info.num_subcores
    NW = NC * NS
    assert B % (8 * NW) == 0 and D % info.num_lanes == 0
    b_per_w = B // NW
    mesh = plsc.VectorSubcoreMesh(core_axis_name="c", subcore_axis_name="s")

    half = b_per_w // 2

    @functools.partial(
        pl.kernel,
        mesh=mesh,
        out_type=jax.ShapeDtypeStruct((B, D), jnp.float32),
        scratch_types=[
            pltpu.VMEM((b_per_w,), jnp.int32),
            pltpu.VMEM((half, D), jnp.float32),
            pltpu.VMEM((half, D), jnp.float32),
            pltpu.SemaphoreType.DMA,
            pltpu.SemaphoreType.DMA,
            pltpu.SemaphoreType.DMA,
        ],
    )
    def sc_gather(table_hbm, idx_hbm, out_hbm, idx_v, rows0_v, rows1_v,
                  s0, s1, sw):
        wid = lax.axis_index("s") * NC + lax.axis_index("c")
        base = wid * b_per_w
        pltpu.sync_copy(idx_hbm.at[pl.ds(base, b_per_w)], idx_v)
        c0 = pltpu.async_copy(table_hbm.at[idx_v.at[pl.ds(0, half)]],
                              rows0_v, s0)
        c1 = pltpu.async_copy(table_hbm.at[idx_v.at[pl.ds(half, half)]],
                              rows1_v, s1)
        c0.wait()
        w0 = pltpu.async_copy(rows0_v, out_hbm.at[pl.ds(base, half)], sw)
        c1.wait()
        w1 = pltpu.async_copy(rows1_v, out_hbm.at[pl.ds(base + half, half)],
                              sw)
        w0.wait()
        w1.wait()

    return sc_gather


def _mlp_body(g_ref, f_ref, d_ref, w0_ref, bft_ref, w1_ref, b1_ref,
              w2_ref, b2_ref, w3_ref, b3_ref, out_ref):
    bf = jnp.bfloat16
    f32 = jnp.float32
    w0 = w0_ref[...]
    c = 2.0 * w0 + bft_ref[...]
    r0 = jnp.maximum(w0 + c, 0.0).astype(bf)
    hg = jnp.maximum(g_ref[...] + c, 0.0).astype(bf)
    is_p1 = f_ref[...] != 0
    h_stm = jnp.where(is_p1, hg, r0)
    h_nstm = jnp.where(is_p1, r0, hg)
    xcat = jnp.concatenate([h_stm, h_nstm, d_ref[...]], axis=1)
    x1 = lax.dot_general(xcat, w1_ref[...], _DNT, preferred_element_type=f32)
    h1 = jnp.maximum(x1 + b1_ref[...], 0.0)
    h2 = jnp.maximum(
        lax.dot_general(h1.astype(bf), w2_ref[...], _DNT,
                        preferred_element_type=f32) + b2_ref[...], 0.0)
    x3t = lax.dot_general(w3_ref[...], h2.astype(bf), _DNT,
                          preferred_element_type=f32)
    out_ref[...] = jnp.tanh(x3t + b3_ref[...])


def _mlp_call(g, f2d, dense, w0, bft, w1, b1, w2, b2, w3, b3):
    B = g.shape[0]
    H = w1.shape[0]
    TI = w1.shape[1]
    H2 = w2.shape[0]
    DD = dense.shape[1]
    grid = (B // BLK,)
    rep = lambda i: (0, 0)
    return pl.pallas_call(
        _mlp_body,
        grid=grid,
        in_specs=[
            pl.BlockSpec((BLK, FT_DIM), lambda i: (i, 0)),
            pl.BlockSpec((BLK, 1), lambda i: (i, 0)),
            pl.BlockSpec((BLK, DD), lambda i: (i, 0)),
            pl.BlockSpec((1, FT_DIM), rep),
            pl.BlockSpec((1, FT_DIM), rep),
            pl.BlockSpec((H, TI), rep),
            pl.BlockSpec((1, H), rep),
            pl.BlockSpec((H2, H), rep),
            pl.BlockSpec((1, H2), rep),
            pl.BlockSpec((1, H2), rep),
            pl.BlockSpec((1, 1), rep),
        ],
        out_specs=pl.BlockSpec((1, BLK), lambda i: (0, i)),
        out_shape=jax.ShapeDtypeStruct((1, B), jnp.float32),
    )(g, f2d, dense, w0, bft, w1, b1, w2, b2, w3, b3)


def kernel(sparse_batch, dense_batch, W_ft, b_ft, W1, b1, W2, b2, W3, b3):
    B = sparse_batch.shape[0]
    si = sparse_batch.astype(jnp.int32)
    f = si[:, 0]
    m8 = (f < P1_FEATURE_CUTOFF).astype(jnp.int8)[:, None]

    bf = jnp.bfloat16
    dense_bf = dense_batch.astype(bf)
    w1bf = W1.astype(bf)
    w2bf = W2.astype(bf)
    w3bf = W3.astype(bf)
    w0 = W_ft[0:1, :]
    bft = b_ft[None, :]
    b1r = b1[None, :]
    b2r = b2[None, :]
    b3r = b3.reshape(1, 1)

    sc_gather = _make_sc_gather(W_ft.shape[0], FT_DIM, B)
    g = sc_gather(W_ft, f)
    out = _mlp_call(
        g, m8, dense_bf, w0, bft, w1bf, b1r, w2bf, b2r, w3bf, b3r,
    )
    return out[0]
